# baseline (speedup 1.0000x reference)
import jax
import jax.numpy as jnp
import numpy as np
from jax.experimental import pallas as pl
from jax.experimental.pallas import tpu as pltpu

N = 20
IN = 256
EMB = 4
NCOL = 80
BB = 256


def _prep_kernel(e_ref, eT_ref, M0_ref, M1_ref, Emat_ref, bselT_ref,
                 icol_ref, wc_ref, bias_ref):
    e = e_ref[...]
    eT = eT_ref[...]
    A = jnp.maximum(jnp.dot(e, eT, preferred_element_type=jnp.float32), 0.0)
    colmax = jnp.max(A, axis=0, keepdims=True)
    expA = jnp.exp(A - colmax)
    St = expA / jnp.sum(expA, axis=0, keepdims=True)
    stcol = jnp.concatenate([St, St, St, St], axis=1)
    colW0 = jnp.dot(M0_ref[...], Emat_ref[...],
                    preferred_element_type=jnp.float32)
    colW1 = jnp.dot(M1_ref[...], Emat_ref[...],
                    preferred_element_type=jnp.float32)
    icol = icol_ref[...]
    wc_ref[...] = (icol[:, None, :] * colW0[None, :, :]
                   + stcol[:, None, :] * colW1[None, :, :])
    bias_ref[...] = jnp.dot(bselT_ref[...], Emat_ref[...],
                            preferred_element_type=jnp.float32)


def _main_kernel(x_ref, wc_ref, bias_ref, wl_ref, bl_ref, out_ref):
    t = bias_ref[...]
    for m in range(N):
        t = t + jnp.dot(x_ref[m], wc_ref[m],
                        preferred_element_type=jnp.float32)
    r0 = jax.nn.sigmoid(t[:, 0:N])
    r1 = jax.nn.sigmoid(t[:, N:2 * N])
    hc0 = jnp.tanh(t[:, 2 * N:3 * N])
    hc1 = jnp.tanh(t[:, 3 * N:4 * N])
    h0 = jnp.maximum((1.0 - r0) * hc0, 0.0)
    h1 = jnp.maximum((1.0 - r1) * hc1, 0.0)
    out_ref[...] = (h0 * wl_ref[0:1, 0:1] + h1 * wl_ref[0:1, 1:2]
                    + bl_ref[0:1, 0:1])


@jax.jit
def _run(x, e, Wg, bg, Wu, bu, Wl, bl):
    B = x.shape[0]
    eT = e.T
    Wsel = jnp.concatenate([Wg[:, :, :IN, 2:4], Wu[:, :, :IN, :]], axis=-1)
    M0 = jnp.transpose(Wsel[:, 0], (1, 2, 0)).reshape(IN, 4 * EMB)
    M1 = jnp.transpose(Wsel[:, 1], (1, 2, 0)).reshape(IN, 4 * EMB)
    Emat = jnp.zeros((4, EMB, 4, N), x.dtype)
    for oo in range(4):
        Emat = Emat.at[oo, :, oo, :].set(eT)
    Emat = Emat.reshape(4 * EMB, NCOL)
    bselT = jnp.concatenate([bg[:, 2:4], bu], axis=1).T.reshape(1, 4 * EMB)
    icol = jnp.asarray(np.tile(np.eye(N, dtype=np.float32), (1, 4)))

    wc, bias = pl.pallas_call(
        _prep_kernel,
        out_shape=(
            jax.ShapeDtypeStruct((N, IN, NCOL), jnp.float32),
            jax.ShapeDtypeStruct((1, NCOL), jnp.float32),
        ),
    )(e, eT, M0, M1, Emat, bselT, icol)

    x2 = jnp.transpose(x, (1, 0, 2))
    y = pl.pallas_call(
        _main_kernel,
        grid=(B // BB,),
        in_specs=[
            pl.BlockSpec((N, BB, IN), lambda i: (0, i, 0)),
            pl.BlockSpec((N, IN, NCOL), lambda i: (0, 0, 0)),
            pl.BlockSpec((1, NCOL), lambda i: (0, 0)),
            pl.BlockSpec((1, 2), lambda i: (0, 0)),
            pl.BlockSpec((1, 1), lambda i: (0, 0)),
        ],
        out_specs=pl.BlockSpec((BB, N), lambda i: (i, 0)),
        out_shape=jax.ShapeDtypeStruct((B, N), jnp.float32),
        compiler_params=pltpu.CompilerParams(
            dimension_semantics=("arbitrary",),
        ),
    )(x2, wc, bias, Wl, bl.reshape(1, 1))
    return y[:, :, None]


def kernel(x, e, _, Wg, bg, Wu, bu, Wl, bl):
    return _run(x, e, Wg, bg, Wu, bu, Wl, bl)


# speedup vs baseline: 1.1538x; 1.1538x over previous
import jax
import jax.numpy as jnp
import numpy as np
from jax.experimental import pallas as pl
from jax.experimental.pallas import tpu as pltpu

N = 20
IN = 256
EMB = 4
NCOL = 80
BB = 512


def _prep_kernel(e_ref, eT_ref, M0_ref, M1_ref, Emat_ref, bselT_ref,
                 icol_ref, wc_ref, bias_ref):
    e = e_ref[...]
    eT = eT_ref[...]
    A = jnp.maximum(jnp.dot(e, eT, preferred_element_type=jnp.float32), 0.0)
    colmax = jnp.max(A, axis=0, keepdims=True)
    expA = jnp.exp(A - colmax)
    St = expA / jnp.sum(expA, axis=0, keepdims=True)
    stcol = jnp.concatenate([St, St, St, St], axis=1)
    colW0 = jnp.dot(M0_ref[...], Emat_ref[...],
                    preferred_element_type=jnp.float32)
    colW1 = jnp.dot(M1_ref[...], Emat_ref[...],
                    preferred_element_type=jnp.float32)
    icol = icol_ref[...]
    wc_ref[...] = (icol[:, None, :] * colW0[None, :, :]
                   + stcol[:, None, :] * colW1[None, :, :])
    bias_ref[...] = jnp.dot(bselT_ref[...], Emat_ref[...],
                            preferred_element_type=jnp.float32)


def _main_kernel(x_ref, wc_ref, bias_ref, wl_ref, bl_ref, out_ref):
    t = bias_ref[...]
    for m in range(N):
        t = t + jnp.dot(x_ref[m], wc_ref[m],
                        preferred_element_type=jnp.float32)
    r0 = jax.nn.sigmoid(t[:, 0:N])
    r1 = jax.nn.sigmoid(t[:, N:2 * N])
    hc0 = jnp.tanh(t[:, 2 * N:3 * N])
    hc1 = jnp.tanh(t[:, 3 * N:4 * N])
    h0 = jnp.maximum((1.0 - r0) * hc0, 0.0)
    h1 = jnp.maximum((1.0 - r1) * hc1, 0.0)
    out_ref[...] = (h0 * wl_ref[0:1, 0:1] + h1 * wl_ref[0:1, 1:2]
                    + bl_ref[0:1, 0:1])


@jax.jit
def _run(x, e, Wg, bg, Wu, bu, Wl, bl):
    B = x.shape[0]
    eT = e.T
    Wsel = jnp.concatenate([Wg[:, :, :IN, 2:4], Wu[:, :, :IN, :]], axis=-1)
    M0 = jnp.transpose(Wsel[:, 0], (1, 2, 0)).reshape(IN, 4 * EMB)
    M1 = jnp.transpose(Wsel[:, 1], (1, 2, 0)).reshape(IN, 4 * EMB)
    Emat = jnp.zeros((4, EMB, 4, N), x.dtype)
    for oo in range(4):
        Emat = Emat.at[oo, :, oo, :].set(eT)
    Emat = Emat.reshape(4 * EMB, NCOL)
    bselT = jnp.concatenate([bg[:, 2:4], bu], axis=1).T.reshape(1, 4 * EMB)
    icol = jnp.asarray(np.tile(np.eye(N, dtype=np.float32), (1, 4)))

    wc, bias = pl.pallas_call(
        _prep_kernel,
        out_shape=(
            jax.ShapeDtypeStruct((N, IN, NCOL), jnp.float32),
            jax.ShapeDtypeStruct((1, NCOL), jnp.float32),
        ),
    )(e, eT, M0, M1, Emat, bselT, icol)

    x2 = jnp.transpose(x, (1, 0, 2))
    y = pl.pallas_call(
        _main_kernel,
        grid=(B // BB,),
        in_specs=[
            pl.BlockSpec((N, BB, IN), lambda i: (0, i, 0)),
            pl.BlockSpec((N, IN, NCOL), lambda i: (0, 0, 0)),
            pl.BlockSpec((1, NCOL), lambda i: (0, 0)),
            pl.BlockSpec((1, 2), lambda i: (0, 0)),
            pl.BlockSpec((1, 1), lambda i: (0, 0)),
        ],
        out_specs=pl.BlockSpec((BB, N), lambda i: (i, 0)),
        out_shape=jax.ShapeDtypeStruct((B, N), jnp.float32),
        compiler_params=pltpu.CompilerParams(
            dimension_semantics=("parallel",),
        ),
    )(x2, wc, bias, Wl, bl.reshape(1, 1))
    return y[:, :, None]


def kernel(x, e, _, Wg, bg, Wu, bu, Wl, bl):
    return _run(x, e, Wg, bg, Wu, bu, Wl, bl)
